# baseline (speedup 1.0000x reference)
import functools

import jax
import jax.numpy as jnp
from jax import lax
from jax.experimental import pallas as pl
from jax.experimental.pallas import tpu as pltpu
from jax.experimental.pallas import tpu_sc as plsc

N = 10000
E = 320000
D = 128
B = 10
C = 10
PG = N // B
BC = B * C
DE = 16
FCE = B * C * (C - 1)
FCP = 1024

NC = 2
NS = 16
NW = NC * NS
CH = E // NW
LANES = 16
GTH = 128
DUMP = BC
ACC_ROWS = 104
CBUF = CH + 2 * GTH
TRASH = CBUF - LANES
DW = D // 2
XROW = 2 * DW
XSLAB = 1000
NLOADERS = (N // 2) // XSLAB


def _sc_body(src_hbm, dst_hbm, x_hbm, out_hbm,
             src_v, dst_v, srcc, slotc, sidx, slot_idx, hidx, rows_b, rows_v,
             zbuf, shared, xsh, sem, sem2):
    cid = lax.axis_index("c")
    sid = lax.axis_index("s")
    wid = sid * NC + cid
    base = wid * CH
    with jax.named_scope("ph_xstage"):
        @pl.when(sid < NLOADERS)
        def _stage_x():
            pltpu.async_copy(x_hbm.at[pl.ds(sid * XSLAB, XSLAB)],
                             xsh.at[pl.ds(sid * XSLAB, XSLAB)], sem2).wait()
    with jax.named_scope("ph_in_dma"):
        pltpu.sync_copy(src_hbm.at[pl.ds(base, CH)], src_v)
        pltpu.sync_copy(dst_hbm.at[pl.ds(base, CH)], dst_v)

    zero16 = jnp.zeros((LANES,), jnp.float32)

    @pl.when(sid < ACC_ROWS // 8)
    def _zero_shared():
        def zero_row(r, carry):
            for k in range(D // LANES):
                zbuf[r, pl.ds(k * LANES, LANES)] = zero16
            return carry
        lax.fori_loop(0, 8, zero_row, 0)
        pltpu.sync_copy(zbuf.at[pl.ds(0, 8)], shared.at[pl.ds(sid * 8, 8)])

    vPG = jnp.full((LANES,), PG, jnp.int32)
    vC = jnp.full((LANES,), C, jnp.int32)
    vTH = jnp.full((LANES,), PG - C, jnp.int32)
    v0 = jnp.full((LANES,), 0, jnp.int32)
    v1 = jnp.full((LANES,), 1, jnp.int32)
    vTRASH = jnp.full((LANES,), TRASH, jnp.int32)
    vMUL = jnp.full((LANES,), 67109, jnp.int32)
    vSH = jnp.full((LANES,), 26, jnp.int32)

    def scan_body(i, cnt):
        d = dst_v[pl.ds(i * LANES, LANES)]
        s = src_v[pl.ds(i * LANES, LANES)]
        g = lax.shift_right_logical(d * vMUL, vSH)
        r = d - g * vPG
        m = r >= vTH
        slot = g * vC + (r - vTH)
        mi = jnp.where(m, v1, v0)
        cum = plsc.cumsum(mi)
        vcnt = lax.broadcast_in_dim(cnt, (LANES,), ())
        pos = jnp.where(m, vcnt + cum - v1, vTRASH)
        plsc.store_scatter(srcc, [pos], s)
        plsc.store_scatter(slotc, [pos], slot)
        pc = plsc.all_reduce_population_count(m)
        return cnt + pc[0]

    with jax.named_scope("ph_scan"):
        cnt = plsc.parallel_loop(0, CH // LANES, carry=jnp.int32(0),
                                 unroll=8)(scan_body)

    dummy_src = jnp.zeros((LANES,), jnp.int32)
    dummy_slot = jnp.full((LANES,), DUMP, jnp.int32)
    for k in range(GTH // LANES):
        srcc[pl.ds(cnt + k * LANES, LANES)] = dummy_src
        slotc[pl.ds(cnt + k * LANES, LANES)] = dummy_slot

    nch = lax.div(cnt + (GTH - 1), GTH)
    with jax.named_scope("ph_bar1"):
        plsc.subcore_barrier()

    v16 = jnp.full((LANES,), 16, jnp.int32)
    vHI = jnp.full((LANES,), -65536, jnp.int32)
    vONE = jnp.full((LANES,), 1, jnp.int32)

    def conv_row(r, carry):
        sv = sidx[pl.ds(r, LANES)]
        par = (sv[0] & 1) * DW
        for k in range(DW // LANES):
            w = rows_b[r, pl.ds(par + k * LANES, LANES)]
            rows_v[r, pl.ds(k * 32, LANES)] = plsc.bitcast(
                lax.shift_left(w, v16), jnp.float32)
            rows_v[r, pl.ds(k * 32 + LANES, LANES)] = plsc.bitcast(
                w & vHI, jnp.float32)
        return carry

    def chunk(j, carry):
        for k in range(GTH // LANES):
            sv = srcc[pl.ds(j * GTH + k * LANES, LANES)]
            sidx[pl.ds(k * LANES, LANES)] = sv
            hidx[pl.ds(k * LANES, LANES)] = lax.shift_right_logical(sv, vONE)
            slot_idx[pl.ds(k * LANES, LANES)] = slotc[pl.ds(j * GTH + k * LANES, LANES)]
        pltpu.async_copy(xsh.at[hidx], rows_b, sem).wait()
        lax.fori_loop(0, GTH, conv_row, 0)
        pltpu.sync_copy(rows_v, shared.at[slot_idx], add=True)
        return carry

    with jax.named_scope("ph_chunks"):
        lax.fori_loop(0, nch, chunk, 0)
    with jax.named_scope("ph_bar2"):
        plsc.subcore_barrier()

    @pl.when(sid == 0)
    def _write_out():
        pltpu.sync_copy(shared, out_hbm.at[cid])


def _make_sc_aggregate():
    return pl.kernel(
        _sc_body,
        out_type=jax.ShapeDtypeStruct((NC, ACC_ROWS, D), jnp.float32),
        mesh=plsc.VectorSubcoreMesh(
            core_axis_name="c", subcore_axis_name="s",
            num_cores=NC, num_subcores=NS),
        compiler_params=pltpu.CompilerParams(needs_layout_passes=False),
        scratch_types=[
            pltpu.VMEM((CH,), jnp.int32),
            pltpu.VMEM((CH,), jnp.int32),
            pltpu.VMEM((CBUF,), jnp.int32),
            pltpu.VMEM((CBUF,), jnp.int32),
            pltpu.VMEM((GTH + LANES,), jnp.int32),
            pltpu.VMEM((GTH,), jnp.int32),
            pltpu.VMEM((GTH,), jnp.int32),
            pltpu.VMEM((GTH, XROW), jnp.int32),
            pltpu.VMEM((GTH, D), jnp.float32),
            pltpu.VMEM((ACC_ROWS, D), jnp.float32),
            pltpu.VMEM_SHARED((ACC_ROWS, D), jnp.float32),
            pltpu.VMEM_SHARED((N // 2, XROW), jnp.int32),
            pltpu.SemaphoreType.DMA,
            pltpu.SemaphoreType.DMA,
        ],
    )


def _tc_body(parts, xc, fs_col, fd_row, fe, gf,
             W_conv, W_self, b_conv, W_mid, W_edge, b_mid,
             W_inst, b_inst, W_dec, b_dec, out):
    f32 = jnp.float32
    dot = functools.partial(jnp.dot, preferred_element_type=f32,
                            precision=lax.Precision.HIGHEST)

    bf = lambda a: a.astype(jnp.bfloat16).astype(f32)

    p = parts[...]
    agg_p = (p[0] + p[1])[:BC, :]
    pp = lax.broadcasted_iota(jnp.int32, (D, D), 0)
    ff = lax.broadcasted_iota(jnp.int32, (D, D), 1)
    pi = 32 * (ff // 32) + 16 * (ff % 2) + (ff % 32) // 2
    P = (pp == pi).astype(f32)
    agg = dot(agg_p, P)
    h = jnp.maximum(dot(agg, W_conv[...]) + dot(xc[...], W_self[...])
                    + b_conv[...], 0.0)
    hb = bf(h)

    cand_row = lax.broadcasted_iota(jnp.int32, (BC, FCP), 0)
    Mdst = (fd_row[...] == cand_row).astype(f32)
    ef = dot(Mdst, fe[...])
    cand_col = lax.broadcasted_iota(jnp.int32, (FCP, BC), 1)
    MsrcT = (fs_col[...] == cand_col).astype(f32)
    A = dot(Mdst, MsrcT)
    cand2 = jnp.maximum(dot(dot(A, hb), W_mid[...])
                        + dot(ef, W_edge[...]) + b_mid[...], 0.0)
    cand2b = bf(cand2)

    gi = jnp.maximum(dot(gf[...], W_inst[...]) + b_inst[...], 0.0)
    gib = bf(gi)
    grow = lax.broadcasted_iota(jnp.int32, (BC, B), 0) // C
    gcol = lax.broadcasted_iota(jnp.int32, (BC, B), 1)
    GG = (grow == gcol).astype(f32)
    grep = dot(GG, gib)

    Wd = W_dec[...]
    logits = (dot(hb, Wd[0:D, :]) + dot(cand2b, Wd[D:2 * D, :])
              + dot(grep, Wd[2 * D:3 * D, :]) + b_dec[...])

    ga = lax.broadcasted_iota(jnp.int32, (B, BC), 1) // C
    gr = lax.broadcasted_iota(jnp.int32, (B, BC), 0)
    G10 = (ga == gr).astype(f32)
    ja = lax.broadcasted_iota(jnp.int32, (BC, C), 0) % C
    jc = lax.broadcasted_iota(jnp.int32, (BC, C), 1)
    J = (ja == jc).astype(f32)
    L = dot(G10, logits * J)

    mx = jnp.max(L, axis=1, keepdims=True)
    ex = jnp.exp(L - mx)
    out[...] = ex / jnp.sum(ex, axis=1, keepdims=True)


def kernel(x, edge_index, fc_edge_index, fc_edge_feat, globalFeat,
           W_conv, W_self, b_conv, W_mid, W_edge, b_mid,
           W_inst, b_inst, W_dec, b_dec):
    rb = lambda a: lax.optimization_barrier(
        a.astype(jnp.bfloat16)).astype(jnp.float32)
    xb16 = x.astype(jnp.bfloat16)
    xi = lax.bitcast_convert_type(xb16.reshape(N // 2, XROW, 2), jnp.int32)
    parts = _make_sc_aggregate()(edge_index[0], edge_index[1], xi)

    xc = xb16.reshape(B, PG, D)[:, PG - C:, :].reshape(BC, D).astype(
        jnp.float32)
    fs_col = jnp.pad(fc_edge_index[0], (0, FCP - FCE),
                     constant_values=-1).reshape(FCP, 1)
    fd_row = jnp.pad(fc_edge_index[1], (0, FCP - FCE),
                     constant_values=-1).reshape(1, FCP)
    fe = jnp.pad(rb(fc_edge_feat), ((0, FCP - FCE), (0, 0)))

    return pl.pallas_call(
        _tc_body,
        out_shape=jax.ShapeDtypeStruct((B, C), jnp.float32),
    )(parts, xc, fs_col, fd_row, fe, rb(globalFeat),
      rb(W_conv), rb(W_self), b_conv.reshape(1, D), rb(W_mid), rb(W_edge),
      b_mid.reshape(1, D), rb(W_inst), b_inst.reshape(1, D),
      rb(W_dec), b_dec.reshape(1, 1))


# speedup vs baseline: 8.8188x; 8.8188x over previous
import functools

import jax
import jax.numpy as jnp
from jax import lax
from jax.experimental import pallas as pl
from jax.experimental.pallas import tpu as pltpu
from jax.experimental.pallas import tpu_sc as plsc

N = 10000
E = 320000
D = 128
B = 10
C = 10
PG = N // B
BC = B * C
DE = 16
FCE = B * C * (C - 1)
FCP = 1024

NC = 2
NS = 16
NW = NC * NS
CH = E // NW
LANES = 16
GTH = 128
DUMP = BC
ACC_ROWS = 104
CBUF = CH + 2 * GTH
TRASH = CBUF - LANES
DW = D // 2
XROW = 2 * DW
XSLAB = 1000
NLOADERS = (N // 2) // XSLAB


def _sc_body(src_hbm, dst_hbm, x_hbm, out_hbm,
             src_v, dst_v, srcc, slotc, sidx, slot_idx, hidx, rows_b, rows_v,
             zbuf, shared, xsh, sem, sem2):
    cid = lax.axis_index("c")
    sid = lax.axis_index("s")
    wid = sid * NC + cid
    base = wid * CH
    with jax.named_scope("ph_xstage"):
        @pl.when(sid < NLOADERS)
        def _stage_x():
            pltpu.async_copy(x_hbm.at[pl.ds(sid * XSLAB, XSLAB)],
                             xsh.at[pl.ds(sid * XSLAB, XSLAB)], sem2).wait()
    with jax.named_scope("ph_in_dma"):
        pltpu.sync_copy(src_hbm.at[pl.ds(base, CH)], src_v)
        pltpu.sync_copy(dst_hbm.at[pl.ds(base, CH)], dst_v)

    zero16 = jnp.zeros((LANES,), jnp.float32)

    @pl.when(sid < ACC_ROWS // 8)
    def _zero_shared():
        def zero_row(r, carry):
            for k in range(D // LANES):
                zbuf[r, pl.ds(k * LANES, LANES)] = zero16
            return carry
        lax.fori_loop(0, 8, zero_row, 0)
        pltpu.sync_copy(zbuf.at[pl.ds(0, 8)], shared.at[pl.ds(sid * 8, 8)])

    vPG = jnp.full((LANES,), PG, jnp.int32)
    vC = jnp.full((LANES,), C, jnp.int32)
    vTH = jnp.full((LANES,), PG - C, jnp.int32)
    v0 = jnp.full((LANES,), 0, jnp.int32)
    v1 = jnp.full((LANES,), 1, jnp.int32)
    vTRASH = jnp.full((LANES,), TRASH, jnp.int32)
    vMUL = jnp.full((LANES,), 67109, jnp.int32)
    vSH = jnp.full((LANES,), 26, jnp.int32)

    def scan_body(i, cnt):
        d = dst_v[pl.ds(i * LANES, LANES)]
        s = src_v[pl.ds(i * LANES, LANES)]
        g = lax.shift_right_logical(d * vMUL, vSH)
        r = d - g * vPG
        m = r >= vTH
        slot = g * vC + (r - vTH)
        mi = jnp.where(m, v1, v0)
        cum = plsc.cumsum(mi)
        vcnt = lax.broadcast_in_dim(cnt, (LANES,), ())
        pos = jnp.where(m, vcnt + cum - v1, vTRASH)
        plsc.store_scatter(srcc, [pos], s)
        plsc.store_scatter(slotc, [pos], slot)
        pc = plsc.all_reduce_population_count(m)
        return cnt + pc[0]

    with jax.named_scope("ph_scan"):
        cnt = plsc.parallel_loop(0, CH // LANES, carry=jnp.int32(0),
                                 unroll=8)(scan_body)

    dummy_src = jnp.zeros((LANES,), jnp.int32)
    dummy_slot = jnp.full((LANES,), DUMP, jnp.int32)
    for k in range(GTH // LANES):
        srcc[pl.ds(cnt + k * LANES, LANES)] = dummy_src
        slotc[pl.ds(cnt + k * LANES, LANES)] = dummy_slot

    nch = lax.div(cnt + (GTH - 1), GTH)
    with jax.named_scope("ph_bar1"):
        plsc.subcore_barrier()

    v16 = jnp.full((LANES,), 16, jnp.int32)
    vHI = jnp.full((LANES,), -65536, jnp.int32)
    vONE = jnp.full((LANES,), 1, jnp.int32)

    def conv_row(r, carry):
        sv = sidx[pl.ds(r, LANES)]
        sh = (1 - (sv[0] & 1)) * 16
        vsh = lax.broadcast_in_dim(sh, (LANES,), ())
        for k in range(D // LANES):
            w = rows_b[r, pl.ds(k * LANES, LANES)]
            rows_v[r, pl.ds(k * LANES, LANES)] = plsc.bitcast(
                lax.shift_left(w, vsh) & vHI, jnp.float32)
        return carry

    def chunk(j, carry):
        for k in range(GTH // LANES):
            sv = srcc[pl.ds(j * GTH + k * LANES, LANES)]
            sidx[pl.ds(k * LANES, LANES)] = sv
            hidx[pl.ds(k * LANES, LANES)] = lax.shift_right_logical(sv, vONE)
            slot_idx[pl.ds(k * LANES, LANES)] = slotc[pl.ds(j * GTH + k * LANES, LANES)]
        pltpu.async_copy(xsh.at[hidx], rows_b, sem).wait()
        lax.fori_loop(0, GTH, conv_row, 0)
        pltpu.sync_copy(rows_v, shared.at[slot_idx], add=True)
        return carry

    with jax.named_scope("ph_chunks"):
        lax.fori_loop(0, nch, chunk, 0)
    with jax.named_scope("ph_bar2"):
        plsc.subcore_barrier()

    @pl.when(sid == 0)
    def _write_out():
        pltpu.sync_copy(shared, out_hbm.at[cid])


def _make_sc_aggregate():
    return pl.kernel(
        _sc_body,
        out_type=jax.ShapeDtypeStruct((NC, ACC_ROWS, D), jnp.float32),
        mesh=plsc.VectorSubcoreMesh(
            core_axis_name="c", subcore_axis_name="s",
            num_cores=NC, num_subcores=NS),
        compiler_params=pltpu.CompilerParams(needs_layout_passes=False),
        scratch_types=[
            pltpu.VMEM((CH,), jnp.int32),
            pltpu.VMEM((CH,), jnp.int32),
            pltpu.VMEM((CBUF,), jnp.int32),
            pltpu.VMEM((CBUF,), jnp.int32),
            pltpu.VMEM((GTH + LANES,), jnp.int32),
            pltpu.VMEM((GTH,), jnp.int32),
            pltpu.VMEM((GTH,), jnp.int32),
            pltpu.VMEM((GTH, XROW), jnp.int32),
            pltpu.VMEM((GTH, D), jnp.float32),
            pltpu.VMEM((ACC_ROWS, D), jnp.float32),
            pltpu.VMEM_SHARED((ACC_ROWS, D), jnp.float32),
            pltpu.VMEM_SHARED((N // 2, XROW), jnp.int32),
            pltpu.SemaphoreType.DMA,
            pltpu.SemaphoreType.DMA,
        ],
    )


def _tc_body(parts, xc, fs_col, fd_row, fe, gf,
             W_conv, W_self, b_conv, W_mid, W_edge, b_mid,
             W_inst, b_inst, W_dec, b_dec, out):
    f32 = jnp.float32
    dot = functools.partial(jnp.dot, preferred_element_type=f32,
                            precision=lax.Precision.HIGHEST)

    bf = lambda a: a.astype(jnp.bfloat16).astype(f32)

    p = parts[...]
    agg = (p[0] + p[1])[:BC, :]
    h = jnp.maximum(dot(agg, W_conv[...]) + dot(xc[...], W_self[...])
                    + b_conv[...], 0.0)
    hb = bf(h)

    cand_row = lax.broadcasted_iota(jnp.int32, (BC, FCP), 0)
    Mdst = (fd_row[...] == cand_row).astype(f32)
    ef = dot(Mdst, fe[...])
    cand_col = lax.broadcasted_iota(jnp.int32, (FCP, BC), 1)
    MsrcT = (fs_col[...] == cand_col).astype(f32)
    A = dot(Mdst, MsrcT)
    cand2 = jnp.maximum(dot(dot(A, hb), W_mid[...])
                        + dot(ef, W_edge[...]) + b_mid[...], 0.0)
    cand2b = bf(cand2)

    gi = jnp.maximum(dot(gf[...], W_inst[...]) + b_inst[...], 0.0)
    gib = bf(gi)
    grow = lax.broadcasted_iota(jnp.int32, (BC, B), 0) // C
    gcol = lax.broadcasted_iota(jnp.int32, (BC, B), 1)
    GG = (grow == gcol).astype(f32)
    grep = dot(GG, gib)

    Wd = W_dec[...]
    logits = (dot(hb, Wd[0:D, :]) + dot(cand2b, Wd[D:2 * D, :])
              + dot(grep, Wd[2 * D:3 * D, :]) + b_dec[...])

    ga = lax.broadcasted_iota(jnp.int32, (B, BC), 1) // C
    gr = lax.broadcasted_iota(jnp.int32, (B, BC), 0)
    G10 = (ga == gr).astype(f32)
    ja = lax.broadcasted_iota(jnp.int32, (BC, C), 0) % C
    jc = lax.broadcasted_iota(jnp.int32, (BC, C), 1)
    J = (ja == jc).astype(f32)
    L = dot(G10, logits * J)

    mx = jnp.max(L, axis=1, keepdims=True)
    ex = jnp.exp(L - mx)
    out[...] = ex / jnp.sum(ex, axis=1, keepdims=True)


def kernel(x, edge_index, fc_edge_index, fc_edge_feat, globalFeat,
           W_conv, W_self, b_conv, W_mid, W_edge, b_mid,
           W_inst, b_inst, W_dec, b_dec):
    rb = lambda a: lax.optimization_barrier(
        a.astype(jnp.bfloat16)).astype(jnp.float32)
    xb16 = x.astype(jnp.bfloat16)
    xp = xb16.reshape(N // 2, 2, D)
    xe = lax.bitcast_convert_type(xp[:, 0, :], jnp.uint16).astype(jnp.uint32)
    xo = lax.bitcast_convert_type(xp[:, 1, :], jnp.uint16).astype(jnp.uint32)
    xi = lax.bitcast_convert_type(xe | (xo << 16), jnp.int32)
    parts = _make_sc_aggregate()(edge_index[0], edge_index[1], xi)

    xc = xb16.reshape(B, PG, D)[:, PG - C:, :].reshape(BC, D).astype(
        jnp.float32)
    fs_col = jnp.pad(fc_edge_index[0], (0, FCP - FCE),
                     constant_values=-1).reshape(FCP, 1)
    fd_row = jnp.pad(fc_edge_index[1], (0, FCP - FCE),
                     constant_values=-1).reshape(1, FCP)
    fe = jnp.pad(rb(fc_edge_feat), ((0, FCP - FCE), (0, 0)))

    return pl.pallas_call(
        _tc_body,
        out_shape=jax.ShapeDtypeStruct((B, C), jnp.float32),
    )(parts, xc, fs_col, fd_row, fe, rb(globalFeat),
      rb(W_conv), rb(W_self), b_conv.reshape(1, D), rb(W_mid), rb(W_edge),
      b_mid.reshape(1, D), rb(W_inst), b_inst.reshape(1, D),
      rb(W_dec), b_dec.reshape(1, 1))
